# baseline (speedup 1.0000x reference)
import jax
import jax.numpy as jnp
import numpy as np
from jax.experimental import pallas as pl

N = 10000
E = 320000
F = 128
G = 64
EDGE_DIM = 4

_DEG_HIST = np.array([0.0, 100.0, 500.0, 2000.0, 4000.0, 2400.0, 800.0, 200.0], dtype=np.float64)
_bins = np.arange(_DEG_HIST.shape[0], dtype=np.float64)
AVG_DEG_LOG = float((np.log(_bins + 1.0) * _DEG_HIST).sum() / _DEG_HIST.sum())


def _head_kernel(z_ref, e_ref, w1_ref, b1_ref, w2_ref, b2_ref, w3_ref, b3_ref,
                 w4_ref, b4_ref, o_ref):
    z = z_ref[...]
    z = jax.nn.relu(jnp.dot(z, w1_ref[...], preferred_element_type=jnp.float32) + b1_ref[...])
    z = jax.nn.relu(jnp.dot(z, w2_ref[...], preferred_element_type=jnp.float32) + b2_ref[...])
    z = jax.nn.relu(jnp.dot(z, w3_ref[...], preferred_element_type=jnp.float32) + e_ref[...] + b3_ref[...])
    o_ref[...] = jnp.dot(z, w4_ref[...], preferred_element_type=jnp.float32) + b4_ref[...]


def _head(z, intarna_energy, params):
    e_term = intarna_energy[:, None] * params['lin3_W'][-1][None, :]
    return pl.pallas_call(
        _head_kernel,
        out_shape=jax.ShapeDtypeStruct((G, 2), jnp.float32),
    )(z, e_term, params['lin1_W'], params['lin1_b'][None, :],
      params['lin2_W'], params['lin2_b'][None, :],
      params['lin3_W'][:-1], params['lin3_b'][None, :],
      params['lin4_W'], params['lin4_b'][None, :])


def _pna_conv(x, edge_index, edge_attr, p):
    src = edge_index[0]
    dst = edge_index[1]
    e = edge_attr @ p['edge_W'] + p['edge_b']
    h = jnp.concatenate([x[dst], x[src], e], axis=-1)
    m = h @ p['pre_W'] + p['pre_b']
    deg = jax.ops.segment_sum(jnp.ones((m.shape[0],), m.dtype), dst, num_segments=N)
    deg_c = jnp.clip(deg, 1.0, None)[:, None]
    mean = jax.ops.segment_sum(m, dst, num_segments=N) / deg_c
    sq_mean = jax.ops.segment_sum(m * m, dst, num_segments=N) / deg_c
    std = jnp.sqrt(jax.nn.relu(sq_mean - mean * mean) + 1e-5)
    has = (deg > 0)[:, None]
    mx = jnp.where(has, jax.ops.segment_max(m, dst, num_segments=N), 0.0)
    mn = jnp.where(has, -jax.ops.segment_max(-m, dst, num_segments=N), 0.0)
    agg = jnp.concatenate([mean, mn, mx, std], axis=-1)
    amp = jnp.log(jnp.clip(deg, 1.0, None) + 1.0)[:, None] / AVG_DEG_LOG
    out = jnp.concatenate([agg, agg * amp, agg / amp], axis=-1)
    out = jnp.concatenate([x, out], axis=-1)
    out = out @ p['post_W'] + p['post_b']
    return out @ p['lin_W'] + p['lin_b']


def _bn(x, g, b):
    mu = x.mean(axis=0)
    var = x.var(axis=0)
    return (x - mu) / jnp.sqrt(var + 1e-5) * g + b


def _pools(x, batch):
    s = jax.ops.segment_sum(x, batch, num_segments=G)
    cnt = jax.ops.segment_sum(jnp.ones((x.shape[0],), x.dtype), batch, num_segments=G)
    has = (cnt > 0)[:, None]
    mx = jnp.where(has, jax.ops.segment_max(x, batch, num_segments=G), 0.0)
    mean = s / jnp.clip(cnt, 1.0, None)[:, None]
    return s, mx, mean


def kernel(x, edge_index, edge_attr, intarna_energy, batch, dropout_conv_1_2, dropout_conv_rest, params):
    h = x
    xs = []
    for i in range(6):
        p = params['convs'][i]
        h = _pna_conv(h, edge_index, edge_attr, p)
        h = _bn(h, p['bn_gamma'], p['bn_beta'])
        h = jax.nn.relu(h)
        if i < 5:
            s, _, _ = _pools(h, batch)
            xs.append(s)
    s, mx, mean = _pools(h, batch)
    xs.extend([s, mx, mean])
    z = jnp.concatenate(xs, axis=1)
    return _head(z, intarna_energy, params)


# speedup vs baseline: 1.2382x; 1.2382x over previous
import functools

import jax
import jax.numpy as jnp
import numpy as np
from jax.experimental import pallas as pl
from jax.experimental.pallas import tpu as pltpu

N = 10000
E = 320000
F = 128
G = 64

NP_ = 10240
BN = 512
NB_N = NP_ // BN
BE = 3200
NB_E = E // BE

_DEG_HIST = np.array([0.0, 100.0, 500.0, 2000.0, 4000.0, 2400.0, 800.0, 200.0], dtype=np.float64)
_b = np.arange(_DEG_HIST.shape[0], dtype=np.float64)
AVG_DEG_LOG = float((np.log(_b + 1.0) * _DEG_HIST).sum() / _DEG_HIST.sum())
NEG_BIG = -3.0e38


def _agg_kernel(src_hbm, dst_hbm, ea_ref, b3_ref, we_ref, cb_ref,
                add_ref, max_ref, c3_scr, s_smem, d_smem, sem1, sem2):
    k = pl.program_id(0)

    @pl.when(k == 0)
    def _():
        add_ref[...] = jnp.zeros_like(add_ref)
        max_ref[...] = jnp.full_like(max_ref, NEG_BIG)

    cp1 = pltpu.make_async_copy(src_hbm.at[pl.ds(k * BE, BE)], s_smem, sem1)
    cp1.start()
    cp2 = pltpu.make_async_copy(dst_hbm.at[pl.ds(k * BE, BE)], d_smem, sem2)
    cp2.start()
    c2 = jnp.dot(ea_ref[...], we_ref[...],
                 preferred_element_type=jnp.float32) + cb_ref[...]
    c3_scr[...] = c2.reshape(BE, 1, F)
    cp1.wait()
    cp2.wait()

    def body(e, carry):
        s = s_smem[e]
        d = d_smem[e]
        w = b3_ref[pl.ds(s, 1)] + c3_scr[pl.ds(e, 1)]
        addrow = jnp.concatenate([w, w * w, jnp.ones_like(w)], axis=2)
        maxrow = jnp.concatenate([w, -w], axis=2)
        add_ref[pl.ds(d, 1)] = add_ref[pl.ds(d, 1)] + addrow
        max_ref[pl.ds(d, 1)] = jnp.maximum(max_ref[pl.ds(d, 1)], maxrow)
        return carry

    jax.lax.fori_loop(0, BE, body, 0)


def _agg(src, dst, ea8, b3, we8, cbias):
    return pl.pallas_call(
        _agg_kernel,
        grid=(NB_E,),
        in_specs=[
            pl.BlockSpec(memory_space=pl.ANY),
            pl.BlockSpec(memory_space=pl.ANY),
            pl.BlockSpec((BE, 8), lambda k: (k, 0)),
            pl.BlockSpec((NP_, 1, F), lambda k: (0, 0, 0)),
            pl.BlockSpec((8, F), lambda k: (0, 0)),
            pl.BlockSpec((1, F), lambda k: (0, 0)),
        ],
        out_specs=[
            pl.BlockSpec((NP_, 1, 3 * F), lambda k: (0, 0, 0)),
            pl.BlockSpec((NP_, 1, 2 * F), lambda k: (0, 0, 0)),
        ],
        out_shape=[
            jax.ShapeDtypeStruct((NP_, 1, 3 * F), jnp.float32),
            jax.ShapeDtypeStruct((NP_, 1, 2 * F), jnp.float32),
        ],
        scratch_shapes=[
            pltpu.VMEM((BE, 1, F), jnp.float32),
            pltpu.SMEM((BE,), jnp.int32),
            pltpu.SMEM((BE,), jnp.int32),
            pltpu.SemaphoreType.DMA,
            pltpu.SemaphoreType.DMA,
        ],
    )(src, dst, ea8, b3, we8, cbias)


def _post_kernel(add_ref, max_ref, ab_ref, x_ref, pw_ref, pb_ref, lw_ref, lb_ref,
                 h_ref, st_ref):
    i = pl.program_id(0)
    add = add_ref[...]
    sum_bc = add[:, 0:F]
    sumsq_bc = add[:, F:2 * F]
    deg = add[:, 2 * F:3 * F]
    a = ab_ref[:, 0:F]
    mxw = max_ref[:, 0:F]
    mnw = max_ref[:, F:2 * F]
    deg_c = jnp.maximum(deg, 1.0)
    mean_bc = sum_bc / deg_c
    has = deg > 0.0
    mean = jnp.where(has, a + mean_bc, 0.0)
    std = jnp.sqrt(jax.nn.relu(sumsq_bc / deg_c - mean_bc * mean_bc) + 1e-5)
    mx = jnp.where(has, a + mxw, 0.0)
    mn = jnp.where(has, a - mnw, 0.0)
    amp = jnp.log(deg_c + 1.0) * (1.0 / AVG_DEG_LOG)
    inva = 1.0 / amp
    out13 = jnp.concatenate(
        [x_ref[...], mean, mn, mx, std,
         mean * amp, mn * amp, mx * amp, std * amp,
         mean * inva, mn * inva, mx * inva, std * inva], axis=1)
    h = jnp.dot(out13, pw_ref[...], preferred_element_type=jnp.float32) + pb_ref[...]
    h = jnp.dot(h, lw_ref[...], preferred_element_type=jnp.float32) + lb_ref[...]
    rid = jax.lax.broadcasted_iota(jnp.int32, (BN, F), 0) + i * BN
    h = jnp.where(rid < N, h, 0.0)
    h_ref[...] = h
    cs = jnp.sum(h, axis=0, keepdims=True)
    cs2 = jnp.sum(h * h, axis=0, keepdims=True)
    stat = jnp.concatenate([cs, cs2, jnp.zeros((6, F), jnp.float32)], axis=0)
    prev = jnp.where(i == 0, jnp.zeros_like(stat), st_ref[...])
    st_ref[...] = prev + stat


def _post(add_acc, max_acc, ab, x, p):
    return pl.pallas_call(
        _post_kernel,
        grid=(NB_N,),
        in_specs=[
            pl.BlockSpec((BN, 3 * F), lambda i: (i, 0)),
            pl.BlockSpec((BN, 2 * F), lambda i: (i, 0)),
            pl.BlockSpec((BN, 2 * F), lambda i: (i, 0)),
            pl.BlockSpec((BN, F), lambda i: (i, 0)),
            pl.BlockSpec((13 * F, F), lambda i: (0, 0)),
            pl.BlockSpec((1, F), lambda i: (0, 0)),
            pl.BlockSpec((F, F), lambda i: (0, 0)),
            pl.BlockSpec((1, F), lambda i: (0, 0)),
        ],
        out_specs=[
            pl.BlockSpec((BN, F), lambda i: (i, 0)),
            pl.BlockSpec((8, F), lambda i: (0, 0)),
        ],
        out_shape=[
            jax.ShapeDtypeStruct((NP_, F), jnp.float32),
            jax.ShapeDtypeStruct((8, F), jnp.float32),
        ],
    )(add_acc, max_acc, ab, x, p['post_W'], p['post_b'][None, :],
      p['lin_W'], p['lin_b'][None, :])


def _bnact_kernel(act, do_pool, do_max, hp_ref, sc_ref, sh_ref, w12_ref, batch_hbm,
                  *refs):
    if do_pool:
        if do_max:
            h_ref, ab_ref, ps_ref, pm_ref, h3_scr, b_smem, sem = refs
        else:
            h_ref, ab_ref, ps_ref, h3_scr, b_smem, sem = refs
    else:
        h_ref, ab_ref = refs[0], refs[1]
    i = pl.program_id(0)
    if do_pool:
        cp = pltpu.make_async_copy(batch_hbm.at[pl.ds(i * BN, BN)], b_smem, sem)
        cp.start()

        @pl.when(i == 0)
        def _():
            ps_ref[...] = jnp.zeros_like(ps_ref)
            if do_max:
                pm_ref[...] = jnp.zeros_like(pm_ref)

    h = hp_ref[...] * sc_ref[...] + sh_ref[...]
    if act:
        h = jax.nn.relu(h)
    rid = jax.lax.broadcasted_iota(jnp.int32, (BN, F), 0) + i * BN
    h = jnp.where(rid < N, h, 0.0)
    h_ref[...] = h
    ab_ref[...] = jnp.dot(h, w12_ref[...], preferred_element_type=jnp.float32)
    if do_pool:
        h3_scr[...] = h.reshape(BN, 1, F)
        cp.wait()

        def body(n, carry):
            g = b_smem[n]
            row = h3_scr[pl.ds(n, 1)]
            ps_ref[pl.ds(g, 1)] = ps_ref[pl.ds(g, 1)] + row
            if do_max:
                pm_ref[pl.ds(g, 1)] = jnp.maximum(pm_ref[pl.ds(g, 1)], row)
            return carry

        jax.lax.fori_loop(0, BN, body, 0)


def _bnact(hp, scale, shift, w12, batch_pad, act, do_pool, do_max):
    out_specs = [pl.BlockSpec((BN, F), lambda i: (i, 0)),
                 pl.BlockSpec((BN, 2 * F), lambda i: (i, 0))]
    out_shape = [jax.ShapeDtypeStruct((NP_, F), jnp.float32),
                 jax.ShapeDtypeStruct((NP_, 2 * F), jnp.float32)]
    scratch = []
    if do_pool:
        out_specs.append(pl.BlockSpec((G, 1, F), lambda i: (0, 0, 0)))
        out_shape.append(jax.ShapeDtypeStruct((G, 1, F), jnp.float32))
        if do_max:
            out_specs.append(pl.BlockSpec((G, 1, F), lambda i: (0, 0, 0)))
            out_shape.append(jax.ShapeDtypeStruct((G, 1, F), jnp.float32))
        scratch = [pltpu.VMEM((BN, 1, F), jnp.float32),
                   pltpu.SMEM((BN,), jnp.int32), pltpu.SemaphoreType.DMA]
    return pl.pallas_call(
        functools.partial(_bnact_kernel, act, do_pool, do_max),
        grid=(NB_N,),
        in_specs=[
            pl.BlockSpec((BN, F), lambda i: (i, 0)),
            pl.BlockSpec((1, F), lambda i: (0, 0)),
            pl.BlockSpec((1, F), lambda i: (0, 0)),
            pl.BlockSpec((F, 2 * F), lambda i: (0, 0)),
            pl.BlockSpec(memory_space=pl.ANY),
        ],
        out_specs=out_specs,
        out_shape=out_shape,
        scratch_shapes=scratch,
    )(hp, scale, shift, w12, batch_pad)


def _head_kernel(z_ref, e_ref, w1_ref, b1_ref, w2_ref, b2_ref, w3_ref, b3_ref,
                 w4_ref, b4_ref, o_ref):
    z = z_ref[...]
    z = jax.nn.relu(jnp.dot(z, w1_ref[...], preferred_element_type=jnp.float32) + b1_ref[...])
    z = jax.nn.relu(jnp.dot(z, w2_ref[...], preferred_element_type=jnp.float32) + b2_ref[...])
    z = jax.nn.relu(jnp.dot(z, w3_ref[...], preferred_element_type=jnp.float32) + e_ref[...] + b3_ref[...])
    o_ref[...] = jnp.dot(z, w4_ref[...], preferred_element_type=jnp.float32) + b4_ref[...]


def _head(z, intarna_energy, params):
    e_term = intarna_energy[:, None] * params['lin3_W'][-1][None, :]
    return pl.pallas_call(
        _head_kernel,
        out_shape=jax.ShapeDtypeStruct((G, 2), jnp.float32),
    )(z, e_term, params['lin1_W'], params['lin1_b'][None, :],
      params['lin2_W'], params['lin2_b'][None, :],
      params['lin3_W'][:-1], params['lin3_b'][None, :],
      params['lin4_W'], params['lin4_b'][None, :])


def kernel(x, edge_index, edge_attr, intarna_energy, batch, dropout_conv_1_2, dropout_conv_rest, params):
    src = edge_index[0]
    dst = edge_index[1]
    ea8 = jnp.pad(edge_attr, ((0, 0), (0, 4)))
    x_pad = jnp.pad(x, ((0, NP_ - N), (0, 0)))
    batch_pad = jnp.pad(batch, (0, NP_ - N), constant_values=G - 1)
    cnt_g = jax.ops.segment_sum(jnp.ones((N,), jnp.float32), batch, num_segments=G)

    convs = params['convs']

    def layer_mats(p):
        w12 = jnp.concatenate([p['pre_W'][0:F], p['pre_W'][F:2 * F]], axis=1)
        we8 = jnp.pad(p['edge_W'] @ p['pre_W'][2 * F:3 * F], ((0, 4), (0, 0)))
        cbias = (p['edge_b'] @ p['pre_W'][2 * F:3 * F] + p['pre_b'])[None, :]
        return w12, we8, cbias

    ones_sc = jnp.ones((1, F), jnp.float32)
    zeros_sc = jnp.zeros((1, F), jnp.float32)

    w12, we8, cbias = layer_mats(convs[0])
    h, ab = _bnact(x_pad, ones_sc, zeros_sc, w12, batch_pad, False, False, False)

    pools = []
    pool6 = None
    for i in range(6):
        p = convs[i]
        b3 = ab[:, F:].reshape(NP_, 1, F)
        add3, max3 = _agg(src, dst, ea8, b3, we8, cbias)
        add_acc = add3.reshape(NP_, 3 * F)
        max_acc = max3.reshape(NP_, 2 * F)
        h_pre, stats = _post(add_acc, max_acc, ab, h, p)
        mu = stats[0:1] / N
        var = stats[1:2] / N - mu * mu
        scale = p['bn_gamma'][None, :] / jnp.sqrt(var + 1e-5)
        shift = p['bn_beta'][None, :] - mu * scale
        if i < 5:
            w12, we8, cbias = layer_mats(convs[i + 1])
            h, ab, psum = _bnact(h_pre, scale, shift, w12, batch_pad, True, True, False)
            pools.append(psum.reshape(G, F))
        else:
            h, ab, psum, pmax = _bnact(h_pre, scale, shift, w12, batch_pad, True, True, True)
            psum = psum.reshape(G, F)
            pmax = pmax.reshape(G, F)
            pmean = psum / jnp.clip(cnt_g, 1.0, None)[:, None]
            pool6 = (psum, pmax, pmean)

    z = jnp.concatenate(pools + list(pool6), axis=1)
    return _head(z, intarna_energy, params)
